# baseline (speedup 1.0000x reference)
import jax
import jax.numpy as jnp
from jax import lax
from jax.experimental import pallas as pl
from jax.experimental.pallas import tpu as pltpu
from jax.experimental.pallas import tpu_sc as plsc

NC = 2
NS = 16
NW = NC * NS
CH = 128
BLK = 8
EBLK = CH * BLK


def _copy_idx(ibuf, off, dstv):
    for k in range(0, CH, 16):
        dstv[pl.ds(k, 16)] = ibuf[pl.ds(off + k, 16)]


def _make_sc_agg(n_nodes, d, n_edges, frac0=0.5):
    rpt = n_nodes // NS
    nch = n_edges // CH
    per_pair = nch // NS
    cpt0 = int(round(frac0 * per_pair / BLK)) * BLK
    cpt1 = per_pair - cpt0
    mesh = plsc.VectorSubcoreMesh(core_axis_name="c", subcore_axis_name="s",
                                  num_cores=NC, num_subcores=NS)

    def body(h_hbm, src_hbm, dst_hbm, zero_hbm, agg_out,
             isrc, idst, dstv0, dstv1, rows0, rows1, agg_sh,
             sem0, sem1):
        cid = lax.axis_index("c")
        sid = lax.axis_index("s")
        wid = sid * NC + cid
        r0 = sid * rpt
        dstv = (dstv0, dstv1)
        rows = (rows0, rows1)
        sems = (sem0, sem1)

        for k in range(0, rpt, CH):
            sz = min(CH, rpt - k)
            pltpu.sync_copy(zero_hbm.at[pl.ds(r0 + k, sz)],
                            rows0.at[pl.ds(0, sz)])
            pltpu.sync_copy(rows0.at[pl.ds(0, sz)],
                            agg_sh.at[pl.ds(r0 + k, sz)])
        plsc.subcore_barrier()

        tile_e0 = (sid * per_pair + cid * cpt0) * CH

        def block_at(e0):
            pltpu.sync_copy(src_hbm.at[pl.ds(e0, EBLK)], isrc)
            pltpu.sync_copy(dst_hbm.at[pl.ds(e0, EBLK)], idst)
            _copy_idx(idst, 0, dstv[0])
            pend = pltpu.async_copy(
                h_hbm.at[isrc.at[pl.ds(0, CH)]], rows[0], sems[0])
            for j in range(BLK):
                p = j & 1
                q = 1 - p
                cur = pend
                if j + 1 < BLK:
                    _copy_idx(idst, (j + 1) * CH, dstv[q])
                    pend = pltpu.async_copy(
                        h_hbm.at[isrc.at[pl.ds((j + 1) * CH, CH)]],
                        rows[q], sems[q])
                cur.wait()
                pltpu.sync_copy(rows[p], agg_sh.at[dstv[p]], add=True)

        def block(b, carry):
            block_at(tile_e0 + b * EBLK)
            return carry

        if cpt0 == cpt1:
            lax.fori_loop(0, cpt0 // BLK, block, 0)
        else:
            @pl.when(cid == 0)
            def _():
                lax.fori_loop(0, cpt0 // BLK, block, 0)

            @pl.when(cid == 1)
            def _():
                lax.fori_loop(0, cpt1 // BLK, block, 0)
        plsc.subcore_barrier()

        for k in range(0, rpt, CH):
            sz = min(CH, rpt - k)
            pltpu.sync_copy(agg_sh.at[pl.ds(r0 + k, sz)],
                            rows0.at[pl.ds(0, sz)])
            pltpu.sync_copy(rows0.at[pl.ds(0, sz)],
                            agg_out.at[cid, pl.ds(r0 + k, sz)])

    return pl.kernel(
        body,
        out_type=jax.ShapeDtypeStruct((NC, n_nodes, d), jnp.float32),
        mesh=mesh,
        scratch_types=[
            pltpu.VMEM((EBLK,), jnp.int32),
            pltpu.VMEM((EBLK,), jnp.int32),
            pltpu.VMEM((CH,), jnp.int32),
            pltpu.VMEM((CH,), jnp.int32),
            pltpu.VMEM((CH, d), jnp.float32),
            pltpu.VMEM((CH, d), jnp.float32),
            pltpu.VMEM_SHARED((n_nodes, d), jnp.float32),
            pltpu.SemaphoreType.DMA,
            pltpu.SemaphoreType.DMA,
        ])


def _make_sc_deg(n_nodes, d, n_edges):
    rpt = n_nodes // NS
    nch = n_edges // CH
    cpt = nch // NW
    nblk = cpt // BLK
    mesh = plsc.VectorSubcoreMesh(core_axis_name="c", subcore_axis_name="s",
                                  num_cores=NC, num_subcores=NS)

    def body(dst_hbm, zero_hbm, ones_hbm, deg_out,
             idst, dstv0, dstv1, ones, buf, deg_sh):
        cid = lax.axis_index("c")
        sid = lax.axis_index("s")
        wid = sid * NC + cid
        r0 = sid * rpt
        dstv = (dstv0, dstv1)

        for k in range(0, rpt, CH):
            sz = min(CH, rpt - k)
            pltpu.sync_copy(zero_hbm.at[pl.ds(r0 + k, sz)],
                            buf.at[pl.ds(0, sz)])
            pltpu.sync_copy(buf.at[pl.ds(0, sz)],
                            deg_sh.at[pl.ds(r0 + k, sz)])
        pltpu.sync_copy(ones_hbm, ones)
        plsc.subcore_barrier()

        tile_e0 = wid * (cpt * CH)

        def block(b, carry):
            e0 = tile_e0 + b * EBLK
            pltpu.sync_copy(dst_hbm.at[pl.ds(e0, EBLK)], idst)
            for j in range(BLK):
                p = j & 1
                _copy_idx(idst, j * CH, dstv[p])
                pltpu.sync_copy(ones, deg_sh.at[dstv[p]], add=True)
            return carry

        lax.fori_loop(0, nblk, block, 0)
        plsc.subcore_barrier()

        for k in range(0, rpt, CH):
            sz = min(CH, rpt - k)
            pltpu.sync_copy(deg_sh.at[pl.ds(r0 + k, sz)],
                            buf.at[pl.ds(0, sz)])
            pltpu.sync_copy(buf.at[pl.ds(0, sz)],
                            deg_out.at[cid, pl.ds(r0 + k, sz)])

    return pl.kernel(
        body,
        out_type=jax.ShapeDtypeStruct((NC, n_nodes, d), jnp.float32),
        mesh=mesh,
        scratch_types=[
            pltpu.VMEM((EBLK,), jnp.int32),
            pltpu.VMEM((CH,), jnp.int32),
            pltpu.VMEM((CH,), jnp.int32),
            pltpu.VMEM((CH, d), jnp.float32),
            pltpu.VMEM((CH, d), jnp.float32),
            pltpu.VMEM_SHARED((n_nodes, d), jnp.float32),
        ])


def _leaky(x):
    return jnp.where(x >= 0, x, 0.1 * x)


def _encode_body(x_ref, w_ref, b_ref, o_ref):
    h = jnp.dot(x_ref[...], w_ref[...],
                preferred_element_type=jnp.float32) + b_ref[...]
    o_ref[...] = _leaky(h)


def _sage_body(h_ref, agg_ref, deg_ref, wl_ref, bl_ref, wr_ref, o_ref):
    agg = agg_ref[0] + agg_ref[1]
    deg = deg_ref[0, :, 0:1] + deg_ref[1, :, 0:1]
    mean = agg / jnp.maximum(deg, 1.0)
    out = (jnp.dot(mean, wl_ref[...], preferred_element_type=jnp.float32)
           + bl_ref[...]
           + jnp.dot(h_ref[...], wr_ref[...],
                     preferred_element_type=jnp.float32))
    o_ref[...] = _leaky(out)


def _tc_encode(x, w, b, br):
    n, d = x.shape
    return pl.pallas_call(
        _encode_body,
        grid=(n // br,),
        in_specs=[
            pl.BlockSpec((br, d), lambda i: (i, 0)),
            pl.BlockSpec((d, d), lambda i: (0, 0)),
            pl.BlockSpec((1, d), lambda i: (0, 0)),
        ],
        out_specs=pl.BlockSpec((br, d), lambda i: (i, 0)),
        out_shape=jax.ShapeDtypeStruct((n, d), jnp.float32),
    )(x, w, b.reshape(1, d))


def _tc_sage(h, agg_parts, deg_parts, wl, bl, wr, br):
    n, d = h.shape
    return pl.pallas_call(
        _sage_body,
        grid=(n // br,),
        in_specs=[
            pl.BlockSpec((br, d), lambda i: (i, 0)),
            pl.BlockSpec((NC, br, d), lambda i: (0, i, 0)),
            pl.BlockSpec((NC, br, d), lambda i: (0, i, 0)),
            pl.BlockSpec((d, d), lambda i: (0, 0)),
            pl.BlockSpec((1, d), lambda i: (0, 0)),
            pl.BlockSpec((d, d), lambda i: (0, 0)),
        ],
        out_specs=pl.BlockSpec((br, d), lambda i: (i, 0)),
        out_shape=jax.ShapeDtypeStruct((n, d), jnp.float32),
    )(h, agg_parts, deg_parts, wl, bl.reshape(1, d), wr)


def kernel(x, edge_index, W_enc, b_enc, Wl0, bl0, Wr0, Wl1, bl1, Wr1):
    n, d = x.shape
    e = edge_index.shape[1]
    src = edge_index[0].astype(jnp.int32)
    dst = edge_index[1].astype(jnp.int32)

    np_ = -(-n // (NS * 8)) * (NS * 8)
    xp = jnp.pad(x, ((0, np_ - n), (0, 0)))

    per_tile = -(-(-(-e // CH) // NW) // BLK) * BLK
    ep = per_tile * NW * CH
    src = jnp.pad(src, (0, ep - e))
    trash = n + (jnp.arange(ep - e, dtype=jnp.int32) % (np_ - n))
    dst = jnp.concatenate([dst, trash])

    zeros = jnp.zeros((np_, d), jnp.float32)
    ones = jnp.ones((CH, d), jnp.float32)

    sc_agg = _make_sc_agg(np_, d, ep, frac0=0.7)
    sc_deg = _make_sc_deg(np_, d, ep)

    br = np_ // 16
    deg = sc_deg(dst, zeros, ones)
    h0 = _tc_encode(xp, W_enc, b_enc, br)
    agg0 = sc_agg(h0, src, dst, zeros)
    h1 = _tc_sage(h0, agg0, deg, Wl0, bl0, Wr0, br)
    agg1 = sc_agg(h1, src, dst, zeros)
    h2 = _tc_sage(h1, agg1, deg, Wl1, bl1, Wr1, br)
    return h2[:n]


# speedup vs baseline: 1.0556x; 1.0556x over previous
import jax
import jax.numpy as jnp
from jax import lax
from jax.experimental import pallas as pl
from jax.experimental.pallas import tpu as pltpu
from jax.experimental.pallas import tpu_sc as plsc

NC = 2
NS = 16
NW = NC * NS
CH = 128
BLK = 8
EBLK = CH * BLK


def _copy_idx(ibuf, off, dstv):
    for k in range(0, CH, 16):
        dstv[pl.ds(k, 16)] = ibuf[pl.ds(off + k, 16)]


def _make_sc_agg(n_nodes, d, n_edges, frac0=0.5):
    rpt = n_nodes // NS
    nch = n_edges // CH
    per_pair = nch // NS
    cpt0 = int(round(frac0 * per_pair / BLK)) * BLK
    cpt1 = per_pair - cpt0
    mesh = plsc.VectorSubcoreMesh(core_axis_name="c", subcore_axis_name="s",
                                  num_cores=NC, num_subcores=NS)

    def body(h_hbm, src_hbm, dst_hbm, zero_hbm, agg_out,
             isrc, idst, dstv0, dstv1, rows0, rows1, agg_sh,
             sem0, sem1):
        cid = lax.axis_index("c")
        sid = lax.axis_index("s")
        wid = sid * NC + cid
        r0 = sid * rpt
        dstv = (dstv0, dstv1)
        rows = (rows0, rows1)
        sems = (sem0, sem1)

        for k in range(0, rpt, CH):
            sz = min(CH, rpt - k)
            pltpu.sync_copy(zero_hbm.at[pl.ds(r0 + k, sz)],
                            rows0.at[pl.ds(0, sz)])
            pltpu.sync_copy(rows0.at[pl.ds(0, sz)],
                            agg_sh.at[pl.ds(r0 + k, sz)])
        plsc.subcore_barrier()

        tile_e0 = (sid * per_pair + cid * cpt0) * CH

        def block_at(e0):
            pltpu.sync_copy(src_hbm.at[pl.ds(e0, EBLK)], isrc)
            pltpu.sync_copy(dst_hbm.at[pl.ds(e0, EBLK)], idst)
            _copy_idx(idst, 0, dstv[0])
            pend = pltpu.async_copy(
                h_hbm.at[isrc.at[pl.ds(0, CH)]], rows[0], sems[0])
            for j in range(BLK):
                p = j & 1
                q = 1 - p
                cur = pend
                if j + 1 < BLK:
                    _copy_idx(idst, (j + 1) * CH, dstv[q])
                    pend = pltpu.async_copy(
                        h_hbm.at[isrc.at[pl.ds((j + 1) * CH, CH)]],
                        rows[q], sems[q])
                cur.wait()
                pltpu.sync_copy(rows[p], agg_sh.at[dstv[p]], add=True)

        def block(b, carry):
            block_at(tile_e0 + b * EBLK)
            return carry

        if cpt0 == cpt1:
            lax.fori_loop(0, cpt0 // BLK, block, 0)
        else:
            @pl.when(cid == 0)
            def _():
                lax.fori_loop(0, cpt0 // BLK, block, 0)

            @pl.when(cid == 1)
            def _():
                lax.fori_loop(0, cpt1 // BLK, block, 0)
        plsc.subcore_barrier()

        for k in range(0, rpt, CH):
            sz = min(CH, rpt - k)
            pltpu.sync_copy(agg_sh.at[pl.ds(r0 + k, sz)],
                            rows0.at[pl.ds(0, sz)])
            pltpu.sync_copy(rows0.at[pl.ds(0, sz)],
                            agg_out.at[cid, pl.ds(r0 + k, sz)])

    return pl.kernel(
        body,
        out_type=jax.ShapeDtypeStruct((NC, n_nodes, d), jnp.float32),
        mesh=mesh,
        scratch_types=[
            pltpu.VMEM((EBLK,), jnp.int32),
            pltpu.VMEM((EBLK,), jnp.int32),
            pltpu.VMEM((CH,), jnp.int32),
            pltpu.VMEM((CH,), jnp.int32),
            pltpu.VMEM((CH, d), jnp.float32),
            pltpu.VMEM((CH, d), jnp.float32),
            pltpu.VMEM_SHARED((n_nodes, d), jnp.float32),
            pltpu.SemaphoreType.DMA,
            pltpu.SemaphoreType.DMA,
        ])


def _make_sc_deg(n_nodes, d, n_edges):
    rpt = n_nodes // NS
    nch = n_edges // CH
    cpt = nch // NW
    nblk = cpt // BLK
    mesh = plsc.VectorSubcoreMesh(core_axis_name="c", subcore_axis_name="s",
                                  num_cores=NC, num_subcores=NS)

    def body(dst_hbm, zero_hbm, ones_hbm, deg_out,
             idst, dstv0, dstv1, ones, buf, deg_sh):
        cid = lax.axis_index("c")
        sid = lax.axis_index("s")
        wid = sid * NC + cid
        r0 = sid * rpt
        dstv = (dstv0, dstv1)

        for k in range(0, rpt, CH):
            sz = min(CH, rpt - k)
            pltpu.sync_copy(zero_hbm.at[pl.ds(r0 + k, sz)],
                            buf.at[pl.ds(0, sz)])
            pltpu.sync_copy(buf.at[pl.ds(0, sz)],
                            deg_sh.at[pl.ds(r0 + k, sz)])
        pltpu.sync_copy(ones_hbm, ones)
        plsc.subcore_barrier()

        tile_e0 = wid * (cpt * CH)

        def block(b, carry):
            e0 = tile_e0 + b * EBLK
            pltpu.sync_copy(dst_hbm.at[pl.ds(e0, EBLK)], idst)
            for j in range(BLK):
                p = j & 1
                _copy_idx(idst, j * CH, dstv[p])
                pltpu.sync_copy(ones, deg_sh.at[dstv[p]], add=True)
            return carry

        lax.fori_loop(0, nblk, block, 0)
        plsc.subcore_barrier()

        for k in range(0, rpt, CH):
            sz = min(CH, rpt - k)
            pltpu.sync_copy(deg_sh.at[pl.ds(r0 + k, sz)],
                            buf.at[pl.ds(0, sz)])
            pltpu.sync_copy(buf.at[pl.ds(0, sz)],
                            deg_out.at[cid, pl.ds(r0 + k, sz)])

    return pl.kernel(
        body,
        out_type=jax.ShapeDtypeStruct((NC, n_nodes, d), jnp.float32),
        mesh=mesh,
        scratch_types=[
            pltpu.VMEM((EBLK,), jnp.int32),
            pltpu.VMEM((CH,), jnp.int32),
            pltpu.VMEM((CH,), jnp.int32),
            pltpu.VMEM((CH, d), jnp.float32),
            pltpu.VMEM((CH, d), jnp.float32),
            pltpu.VMEM_SHARED((n_nodes, d), jnp.float32),
        ])


def _leaky(x):
    return jnp.where(x >= 0, x, 0.1 * x)


def _encode_body(x_ref, w_ref, b_ref, o_ref):
    h = jnp.dot(x_ref[...], w_ref[...],
                preferred_element_type=jnp.float32) + b_ref[...]
    o_ref[...] = _leaky(h)


def _sage_body(h_ref, agg_ref, deg_ref, wl_ref, bl_ref, wr_ref, o_ref):
    agg = agg_ref[0] + agg_ref[1]
    deg = deg_ref[0, :, 0:1] + deg_ref[1, :, 0:1]
    mean = agg / jnp.maximum(deg, 1.0)
    out = (jnp.dot(mean, wl_ref[...], preferred_element_type=jnp.float32)
           + bl_ref[...]
           + jnp.dot(h_ref[...], wr_ref[...],
                     preferred_element_type=jnp.float32))
    o_ref[...] = _leaky(out)


def _tc_encode(x, w, b, br):
    n, d = x.shape
    return pl.pallas_call(
        _encode_body,
        grid=(n // br,),
        in_specs=[
            pl.BlockSpec((br, d), lambda i: (i, 0)),
            pl.BlockSpec((d, d), lambda i: (0, 0)),
            pl.BlockSpec((1, d), lambda i: (0, 0)),
        ],
        out_specs=pl.BlockSpec((br, d), lambda i: (i, 0)),
        out_shape=jax.ShapeDtypeStruct((n, d), jnp.float32),
    )(x, w, b.reshape(1, d))


def _tc_sage(h, agg_parts, deg_parts, wl, bl, wr, br):
    n, d = h.shape
    return pl.pallas_call(
        _sage_body,
        grid=(n // br,),
        in_specs=[
            pl.BlockSpec((br, d), lambda i: (i, 0)),
            pl.BlockSpec((NC, br, d), lambda i: (0, i, 0)),
            pl.BlockSpec((NC, br, d), lambda i: (0, i, 0)),
            pl.BlockSpec((d, d), lambda i: (0, 0)),
            pl.BlockSpec((1, d), lambda i: (0, 0)),
            pl.BlockSpec((d, d), lambda i: (0, 0)),
        ],
        out_specs=pl.BlockSpec((br, d), lambda i: (i, 0)),
        out_shape=jax.ShapeDtypeStruct((n, d), jnp.float32),
    )(h, agg_parts, deg_parts, wl, bl.reshape(1, d), wr)


def kernel(x, edge_index, W_enc, b_enc, Wl0, bl0, Wr0, Wl1, bl1, Wr1):
    n, d = x.shape
    e = edge_index.shape[1]
    src = edge_index[0].astype(jnp.int32)
    dst = edge_index[1].astype(jnp.int32)

    np_ = -(-n // (NS * 8)) * (NS * 8)
    xp = jnp.pad(x, ((0, np_ - n), (0, 0)))

    per_tile = -(-(-(-e // CH) // NW) // BLK) * BLK
    ep = per_tile * NW * CH
    src = jnp.pad(src, (0, ep - e))
    trash = n + (jnp.arange(ep - e, dtype=jnp.int32) % (np_ - n))
    dst = jnp.concatenate([dst, trash])

    zeros = jnp.zeros((np_, d), jnp.float32)
    ones = jnp.ones((CH, d), jnp.float32)

    sc_agg = _make_sc_agg(np_, d, ep, frac0=0.74)
    sc_deg = _make_sc_deg(np_, d, ep)

    br = np_ // 16
    deg = sc_deg(dst, zeros, ones)
    h0 = _tc_encode(xp, W_enc, b_enc, br)
    agg0 = sc_agg(h0, src, dst, zeros)
    h1 = _tc_sage(h0, agg0, deg, Wl0, bl0, Wr0, br)
    agg1 = sc_agg(h1, src, dst, zeros)
    h2 = _tc_sage(h1, agg1, deg, Wl1, bl1, Wr1, br)
    return h2[:n]
